# baseline (speedup 1.0000x reference)
import functools

import jax
import jax.numpy as jnp
from jax import lax
from jax.experimental import pallas as pl
from jax.experimental.pallas import tpu as pltpu
from jax.experimental.pallas import tpu_sc as plsc

_N = 16384
_C = 100
_S = 30.0
_BT = 4096
_NB = _N // _BT

_NCORES = 2
_NSUB = 16
_NW = _NCORES * _NSUB
_PW = _N // _NW
_LANES = 16


def _tc_body(xt_ref, t_ref, ml_ref, out_ref):
    i = pl.program_id(0)
    xt = xt_ref[...]
    t = t_ref[...]
    mlr = ml_ref[...]
    ri = lax.broadcasted_iota(jnp.int32, (_C, _C), 0)
    ci = lax.broadcasted_iota(jnp.int32, (_C, _C), 1)
    ml = jnp.sum(
        jnp.where(ri == ci, jnp.broadcast_to(mlr, (_C, _C)), 0.0),
        axis=1,
        keepdims=True,
    )
    row = lax.broadcasted_iota(jnp.int32, (_C, _BT), 0)
    onehot = row == t
    logits = xt * _S - jnp.where(onehot, ml * _S, 0.0)
    m = jnp.max(logits, axis=0, keepdims=True)
    se = jnp.sum(jnp.exp(logits - m), axis=0, keepdims=True)
    tgt = jnp.sum(jnp.where(onehot, xt * _S, 0.0), axis=0, keepdims=True)
    part = jnp.sum(m + jnp.log(se) - tgt)

    @pl.when(i == 0)
    def _():
        out_ref[0, 0] = 0.0

    out_ref[0, 0] += part


def _tc_call(x, target, m_list):
    out = pl.pallas_call(
        _tc_body,
        grid=(_NB,),
        in_specs=[
            pl.BlockSpec((_C, _BT), lambda i: (0, i)),
            pl.BlockSpec((1, _BT), lambda i: (0, i)),
            pl.BlockSpec((1, _C), lambda i: (0, 0)),
        ],
        out_specs=pl.BlockSpec(memory_space=pltpu.SMEM),
        out_shape=jax.ShapeDtypeStruct((1, 1), jnp.float32),
        compiler_params=pltpu.CompilerParams(
            dimension_semantics=("arbitrary",),
        ),
    )(x.T, target.reshape(1, _N), m_list.reshape(1, _C))
    return out[0, 0]


_ROWS_PW = _PW // 128


def _sc_body(tgt_hbm, ml_hbm, out_hbm, tgt_v, mg_v, acc_v, sem):
    wid = lax.axis_index("s") * _NCORES + lax.axis_index("c")
    pltpu.sync_copy(tgt_hbm.at[pl.ds(wid * _ROWS_PW, _ROWS_PW)], tgt_v)
    for j in range(_ROWS_PW):
        pltpu.async_copy(ml_hbm.at[tgt_v.at[j]], mg_v.at[j], sem).wait()
    acc = jnp.zeros((_LANES,), jnp.float32)
    for j in range(_ROWS_PW):
        for k in range(128 // _LANES):
            acc = acc + mg_v[j, pl.ds(k * _LANES, _LANES)]
    acc_v[...] = acc
    pltpu.sync_copy(acc_v, out_hbm.at[wid])


@functools.partial(
    pl.kernel,
    mesh=plsc.VectorSubcoreMesh(core_axis_name="c", subcore_axis_name="s"),
    out_type=jax.ShapeDtypeStruct((_NW, _LANES), jnp.float32),
    scratch_types=[
        pltpu.VMEM((_ROWS_PW, 128), jnp.int32),
        pltpu.VMEM((_ROWS_PW, 128), jnp.float32),
        pltpu.VMEM((_LANES,), jnp.float32),
        pltpu.SemaphoreType.DMA,
    ],
)
def _sc_margin_sums(tgt_hbm, ml_hbm, out_hbm, tgt_v, mg_v, acc_v, sem):
    _sc_body(tgt_hbm, ml_hbm, out_hbm, tgt_v, mg_v, acc_v, sem)


def kernel(x, target, m_list):
    msums = _sc_margin_sums(target.reshape(_N // 128, 128), m_list)
    tc_part = _tc_call(x, target, m_list)
    return (tc_part + _S * jnp.sum(msums)) / _N


# speedup vs baseline: 1.0007x; 1.0007x over previous
import functools

import jax
import jax.numpy as jnp
from jax import lax
from jax.experimental import pallas as pl
from jax.experimental.pallas import tpu as pltpu
from jax.experimental.pallas import tpu_sc as plsc

_N = 16384
_C = 100
_S = 30.0
_BT = 4096
_NB = _N // _BT

_NCORES = 2
_NSUB = 16
_NW = _NCORES * _NSUB
_PW = _N // _NW
_LANES = 16


def _tc_body(xt_ref, t_ref, ml_ref, out_ref):
    i = pl.program_id(0)
    xt = xt_ref[...]
    t = t_ref[...]
    mlr = ml_ref[...]
    ri = lax.broadcasted_iota(jnp.int32, (_C, _C), 0)
    ci = lax.broadcasted_iota(jnp.int32, (_C, _C), 1)
    ml = jnp.sum(
        jnp.where(ri == ci, jnp.broadcast_to(mlr, (_C, _C)), 0.0),
        axis=1,
        keepdims=True,
    )
    row = lax.broadcasted_iota(jnp.int32, (_C, _BT), 0)
    onehot = row == t
    logits = xt * _S - jnp.where(onehot, ml * _S, 0.0)
    m = jnp.max(logits, axis=0, keepdims=True)
    se = jnp.sum(jnp.exp(logits - m), axis=0, keepdims=True)
    tgt = jnp.sum(jnp.where(onehot, xt * _S, 0.0), axis=0, keepdims=True)
    part = jnp.sum(m + jnp.log(se) - tgt)

    @pl.when(i == 0)
    def _():
        out_ref[0, 0] = 0.0

    out_ref[0, 0] += part


def _tc_call(x, target, m_list):
    out = pl.pallas_call(
        _tc_body,
        grid=(_NB,),
        in_specs=[
            pl.BlockSpec((_C, _BT), lambda i: (0, i)),
            pl.BlockSpec((1, _BT), lambda i: (0, i)),
            pl.BlockSpec((1, _C), lambda i: (0, 0)),
        ],
        out_specs=pl.BlockSpec(memory_space=pltpu.SMEM),
        out_shape=jax.ShapeDtypeStruct((1, 1), jnp.float32),
        compiler_params=pltpu.CompilerParams(
            dimension_semantics=("arbitrary",),
        ),
    )(x.T, target.reshape(1, _N), m_list.reshape(1, _C))
    return out[0, 0]


_ROWS_PW = _PW // 128


def _sc_body(tgt_hbm, ml_hbm, out_hbm, tgt_v, mg_v, acc_v, sem):
    wid = lax.axis_index("s") * _NCORES + lax.axis_index("c")
    pltpu.sync_copy(tgt_hbm.at[pl.ds(wid * _ROWS_PW, _ROWS_PW)], tgt_v)
    copies = [
        pltpu.async_copy(ml_hbm.at[tgt_v.at[j]], mg_v.at[j], sem)
        for j in range(_ROWS_PW)
    ]
    for c in copies:
        c.wait()
    acc = jnp.zeros((_LANES,), jnp.float32)
    for j in range(_ROWS_PW):
        for k in range(128 // _LANES):
            acc = acc + mg_v[j, pl.ds(k * _LANES, _LANES)]
    acc_v[...] = acc
    pltpu.sync_copy(acc_v, out_hbm.at[wid])


@functools.partial(
    pl.kernel,
    mesh=plsc.VectorSubcoreMesh(core_axis_name="c", subcore_axis_name="s"),
    out_type=jax.ShapeDtypeStruct((_NW, _LANES), jnp.float32),
    scratch_types=[
        pltpu.VMEM((_ROWS_PW, 128), jnp.int32),
        pltpu.VMEM((_ROWS_PW, 128), jnp.float32),
        pltpu.VMEM((_LANES,), jnp.float32),
        pltpu.SemaphoreType.DMA,
    ],
)
def _sc_margin_sums(tgt_hbm, ml_hbm, out_hbm, tgt_v, mg_v, acc_v, sem):
    _sc_body(tgt_hbm, ml_hbm, out_hbm, tgt_v, mg_v, acc_v, sem)


def kernel(x, target, m_list):
    msums = _sc_margin_sums(target.reshape(_N // 128, 128), m_list)
    tc_part = _tc_call(x, target, m_list)
    return (tc_part + _S * jnp.sum(msums)) / _N


# speedup vs baseline: 13.8780x; 13.8676x over previous
import jax
import jax.numpy as jnp
from jax import lax
from jax.experimental import pallas as pl
from jax.experimental.pallas import tpu as pltpu

_N = 16384
_C = 100
_S = 30.0
_BT = 4096
_NB = _N // _BT


def _body(xt_ref, t_ref, ml_ref, out_ref):
    i = pl.program_id(0)
    xt = xt_ref[...]
    t = t_ref[...]
    mlr = ml_ref[...]
    ri = lax.broadcasted_iota(jnp.int32, (_C, _C), 0)
    ci = lax.broadcasted_iota(jnp.int32, (_C, _C), 1)
    ml = jnp.sum(
        jnp.where(ri == ci, jnp.broadcast_to(mlr, (_C, _C)), 0.0),
        axis=1,
        keepdims=True,
    )
    row = lax.broadcasted_iota(jnp.int32, (_C, _BT), 0)
    onehot = row == t
    logits = xt * _S - jnp.where(onehot, ml * _S, 0.0)
    m = jnp.max(logits, axis=0, keepdims=True)
    se = jnp.sum(jnp.exp(logits - m), axis=0, keepdims=True)
    tgt = jnp.sum(jnp.where(onehot, logits, 0.0), axis=0, keepdims=True)
    part = jnp.sum(m + jnp.log(se) - tgt)

    @pl.when(i == 0)
    def _():
        out_ref[0, 0] = 0.0

    out_ref[0, 0] += part

    @pl.when(i == _NB - 1)
    def _():
        out_ref[0, 0] = out_ref[0, 0] / _N


def kernel(x, target, m_list):
    out = pl.pallas_call(
        _body,
        grid=(_NB,),
        in_specs=[
            pl.BlockSpec((_C, _BT), lambda i: (0, i)),
            pl.BlockSpec((1, _BT), lambda i: (0, i)),
            pl.BlockSpec((1, _C), lambda i: (0, 0)),
        ],
        out_specs=pl.BlockSpec(memory_space=pltpu.SMEM),
        out_shape=jax.ShapeDtypeStruct((1, 1), jnp.float32),
        compiler_params=pltpu.CompilerParams(
            dimension_semantics=("arbitrary",),
        ),
    )(x.T, target.reshape(1, _N), m_list.reshape(1, _C))
    return out[0, 0]
